# Initial kernel scaffold; baseline (speedup 1.0000x reference)
import jax
import jax.numpy as jnp
from jax.experimental import pallas as pl


def kernel(x, edge_index, W1, b1, W2, b2, Wfc, bfc):
    raise NotImplementedError("write your pallas kernel here")


# speedup vs baseline: 12.6688x; 12.6688x over previous
import functools

import jax
import jax.numpy as jnp
from jax import lax
from jax.experimental import pallas as pl
from jax.experimental.pallas import tpu as pltpu
from jax.experimental.pallas import tpu_sc as plsc

N = 10000
E = 320000
D_IN = 128
D_HID = 64
D_OUT = 128

NP = 10240
BLK = 1024
CHK = 4096
NCH = 40
EPAD = 2 * NCH * CHK
EPT = EPAD // 32
DUMMY = NP - 1
CPT = 4

_mesh = plsc.VectorSubcoreMesh(core_axis_name="c", subcore_axis_name="s")
_params = pltpu.CompilerParams(needs_layout_passes=False)


@functools.partial(
    pl.kernel,
    mesh=_mesh,
    out_type=jax.ShapeDtypeStruct((32, NP), jnp.float32),
    compiler_params=_params,
    scratch_types=[
        pltpu.VMEM((NP,), jnp.float32),
        pltpu.VMEM((EPT,), jnp.int32),
    ],
)
def _deg_kernel(dst_hbm, out_hbm, acc, dv):
    c = lax.axis_index("c")
    s = lax.axis_index("s")
    t = c * 16 + s
    pltpu.sync_copy(dst_hbm.at[pl.ds(t * EPT, EPT)], dv)
    z = jnp.zeros((16,), jnp.float32)
    ones = jnp.ones((16,), jnp.float32)

    def zbody(i, _):
        acc[pl.ds(i * 16, 16)] = z
        return 0

    lax.fori_loop(0, NP // 16, zbody, 0)

    def body(g, _):
        ov = dv[pl.ds(g * 16, 16)]
        plsc.addupdate_scatter(acc, [ov], ones)
        return 0

    lax.fori_loop(0, EPT // 16, body, 0)
    pltpu.sync_copy(acc, out_hbm.at[t])


@functools.partial(
    pl.kernel,
    mesh=_mesh,
    out_type=jax.ShapeDtypeStruct((2, D_HID, NP), jnp.float32),
    compiler_params=_params,
    scratch_types=[
        [pltpu.VMEM((NP,), jnp.float32)] * CPT,
        [pltpu.VMEM((NP,), jnp.float32)] * CPT,
        pltpu.VMEM((CHK,), jnp.int32),
        pltpu.VMEM((CHK,), jnp.int32),
    ],
)
def _agg_kernel(hsT_hbm, src_hbm, dst_hbm, out_hbm, tbl, acc, sv, dv):
    c = lax.axis_index("c")
    s = lax.axis_index("s")
    z = jnp.zeros((16,), jnp.float32)
    for j in range(CPT):
        pltpu.sync_copy(hsT_hbm.at[s * CPT + j], tbl[j])

    def zbody(i, _):
        for j in range(CPT):
            acc[j][pl.ds(i * 16, 16)] = z
        return 0

    lax.fori_loop(0, NP // 16, zbody, 0)

    def chunk(k, _):
        base = c * (NCH * CHK) + k * CHK
        pltpu.sync_copy(src_hbm.at[pl.ds(base, CHK)], sv)
        pltpu.sync_copy(dst_hbm.at[pl.ds(base, CHK)], dv)

        def body(g, _):
            iv = sv[pl.ds(g * 16, 16)]
            ov = dv[pl.ds(g * 16, 16)]
            for j in range(CPT):
                vals = plsc.load_gather(tbl[j], [iv])
                plsc.addupdate_scatter(acc[j], [ov], vals)
            return 0

        lax.fori_loop(0, CHK // 16, body, 0)
        return 0

    lax.fori_loop(0, NCH, chunk, 0)
    for j in range(CPT):
        pltpu.sync_copy(acc[j], out_hbm.at[c, s * CPT + j])


def _dinv_row(degp):
    return lax.rsqrt(jnp.sum(degp, axis=0, keepdims=True) + 1.0)


def _tc_a_body(w1t_ref, x_ref, degp_ref, out_ref):
    dinv = _dinv_row(degp_ref[...])
    z = lax.dot_general(
        w1t_ref[...], x_ref[...], (((1,), (1,)), ((), ())),
        preferred_element_type=jnp.float32,
    )
    out_ref[...] = z * dinv


def _tc_b_body(hs1_ref, raw_ref, degp_ref, b1_ref, out_ref):
    i = pl.program_id(0)
    dinv = _dinv_row(degp_ref[...])
    h = jnp.maximum(dinv * (hs1_ref[...] + raw_ref[0] + raw_ref[1]) + b1_ref[...],
                    0.0)
    cols = i * BLK + lax.broadcasted_iota(jnp.int32, (1, BLK), 1)
    out_ref[...] = jnp.where(cols < N, dinv * h, 0.0)


def _tc_c_body(hs2_ref, raw_ref, degp_ref, w2t_ref, b2_ref, wfc_ref, bfc_ref,
               out_ref, accum):
    i = pl.program_id(0)
    dinv = _dinv_row(degp_ref[...])
    agg = dinv * (hs2_ref[...] + raw_ref[0] + raw_ref[1])
    z = lax.dot_general(
        w2t_ref[...], agg, (((1,), (0,)), ((), ())),
        preferred_element_type=jnp.float32,
    ) + b2_ref[...]
    r = jnp.maximum(z, 0.0)
    cols = i * BLK + lax.broadcasted_iota(jnp.int32, (1, BLK), 1)
    r = jnp.where(cols < N, r, 0.0)
    part = jnp.sum(r, axis=1, keepdims=True)

    @pl.when(i == 0)
    def _():
        accum[...] = jnp.zeros_like(accum)

    accum[...] += part

    @pl.when(i == pl.num_programs(0) - 1)
    def _():
        g = jnp.sum(accum[...] * wfc_ref[...], axis=0, keepdims=True)
        out_ref[...] = g / jnp.float32(N) + bfc_ref[...]


def kernel(x, edge_index, W1, b1, W2, b2, Wfc, bfc):
    x_pad = jnp.pad(x, ((0, NP - N), (0, 0)))
    pad = jnp.full((EPAD - E,), DUMMY, jnp.int32)
    src_flat = jnp.concatenate([edge_index[0], pad])
    dst_flat = jnp.concatenate([edge_index[1], pad])

    degp = _deg_kernel(dst_flat)

    grid = NP // BLK
    full = lambda shape: pl.BlockSpec(shape, lambda i: (0,) * len(shape))
    colT = pl.BlockSpec((D_HID, BLK), lambda i: (0, i))
    deg_blk = pl.BlockSpec((32, BLK), lambda i: (0, i))
    raw_blk = pl.BlockSpec((2, D_HID, BLK), lambda i: (0, 0, i))

    hs1T = pl.pallas_call(
        _tc_a_body,
        grid=(grid,),
        in_specs=[full((D_HID, D_IN)), pl.BlockSpec((BLK, D_IN), lambda i: (i, 0)),
                  deg_blk],
        out_specs=colT,
        out_shape=jax.ShapeDtypeStruct((D_HID, NP), jnp.float32),
    )(W1.T, x_pad, degp)

    raw1 = _agg_kernel(hs1T, src_flat, dst_flat)

    hs2T = pl.pallas_call(
        _tc_b_body,
        grid=(grid,),
        in_specs=[colT, raw_blk, deg_blk, full((D_HID, 1))],
        out_specs=colT,
        out_shape=jax.ShapeDtypeStruct((D_HID, NP), jnp.float32),
    )(hs1T, raw1, degp, b1.reshape(D_HID, 1))

    raw2 = _agg_kernel(hs2T, src_flat, dst_flat)

    out = pl.pallas_call(
        _tc_c_body,
        grid=(grid,),
        in_specs=[
            colT,
            raw_blk,
            deg_blk,
            full((D_OUT, D_HID)),
            full((D_OUT, 1)),
            full((D_OUT, 1)),
            full((1, 1)),
        ],
        out_specs=pl.BlockSpec((1, 1), lambda i: (0, 0)),
        out_shape=jax.ShapeDtypeStruct((1, 1), jnp.float32),
        scratch_shapes=[pltpu.VMEM((D_OUT, 1), jnp.float32)],
    )(hs2T, raw2, degp, W2.T, b2.reshape(D_OUT, 1), Wfc, bfc.reshape(1, 1))
    return out


# speedup vs baseline: 62.7740x; 4.9550x over previous
import functools

import jax
import jax.numpy as jnp
from jax import lax
from jax.experimental import pallas as pl
from jax.experimental.pallas import tpu as pltpu
from jax.experimental.pallas import tpu_sc as plsc

N = 10000
E = 320000
D_IN = 128
D_HID = 64
D_OUT = 128

NP = 10240
BLK = 1024
CHK = 4096
NCH = 40
EPAD = 2 * NCH * CHK
EPT = EPAD // 32
DUMMY = NP - 1
CPT = 4

_mesh = plsc.VectorSubcoreMesh(core_axis_name="c", subcore_axis_name="s")
_params = pltpu.CompilerParams(needs_layout_passes=False)


@functools.partial(
    pl.kernel,
    mesh=_mesh,
    out_type=jax.ShapeDtypeStruct((32, NP), jnp.float32),
    compiler_params=_params,
    scratch_types=[
        pltpu.VMEM((NP,), jnp.float32),
        pltpu.VMEM((EPT,), jnp.int32),
    ],
)
def _deg_kernel(dst_hbm, out_hbm, acc, dv):
    c = lax.axis_index("c")
    s = lax.axis_index("s")
    t = c * 16 + s
    pltpu.sync_copy(dst_hbm.at[pl.ds(t * EPT, EPT)], dv)
    z = jnp.zeros((16,), jnp.float32)
    ones = jnp.ones((16,), jnp.float32)

    @functools.partial(plsc.parallel_loop, 0, NP // 16, unroll=8)
    def _(i):
        acc[pl.ds(i * 16, 16)] = z

    @functools.partial(plsc.parallel_loop, 0, EPT // 16, unroll=8)
    def _(g):
        ov = dv[pl.ds(g * 16, 16)]
        plsc.addupdate_scatter(acc, [ov], ones)

    pltpu.sync_copy(acc, out_hbm.at[t])


@functools.partial(
    pl.kernel,
    mesh=_mesh,
    out_type=jax.ShapeDtypeStruct((2, D_HID, NP), jnp.float32),
    compiler_params=_params,
    scratch_types=[
        [pltpu.VMEM((NP,), jnp.float32)] * CPT,
        [pltpu.VMEM((NP,), jnp.float32)] * CPT,
        [pltpu.VMEM((CHK,), jnp.int32)] * 2,
        [pltpu.VMEM((CHK,), jnp.int32)] * 2,
        [pltpu.SemaphoreType.DMA] * 2,
        [pltpu.SemaphoreType.DMA] * 2,
    ],
)
def _agg_kernel(hsT_hbm, src_hbm, dst_hbm, out_hbm, tbl, acc, sv, dv, ssem, dsem):
    c = lax.axis_index("c")
    s = lax.axis_index("s")
    z = jnp.zeros((16,), jnp.float32)
    half = c * (NCH * CHK)

    def stage(k, b):
        pltpu.async_copy(src_hbm.at[pl.ds(half + k * CHK, CHK)], sv[b], ssem[b])
        pltpu.async_copy(dst_hbm.at[pl.ds(half + k * CHK, CHK)], dv[b], dsem[b])

    def wait(b):
        pltpu.make_async_copy(src_hbm.at[pl.ds(0, CHK)], sv[b], ssem[b]).wait()
        pltpu.make_async_copy(dst_hbm.at[pl.ds(0, CHK)], dv[b], dsem[b]).wait()

    def process(b):
        @functools.partial(plsc.parallel_loop, 0, CHK // 16, unroll=8)
        def _(g):
            iv = sv[b][pl.ds(g * 16, 16)]
            ov = dv[b][pl.ds(g * 16, 16)]
            for j in range(CPT):
                vals = plsc.load_gather(tbl[j], [iv])
                plsc.addupdate_scatter(acc[j], [ov], vals)

    stage(0, 0)
    for j in range(CPT):
        pltpu.sync_copy(hsT_hbm.at[s * CPT + j], tbl[j])

    @functools.partial(plsc.parallel_loop, 0, NP // 16, unroll=8)
    def _(i):
        for j in range(CPT):
            acc[j][pl.ds(i * 16, 16)] = z

    def pair(p, _):
        wait(0)
        stage(2 * p + 1, 1)
        process(0)
        wait(1)

        @pl.when(p + 1 < NCH // 2)
        def _():
            stage(2 * p + 2, 0)

        process(1)
        return 0

    lax.fori_loop(0, NCH // 2, pair, 0)
    for j in range(CPT):
        pltpu.sync_copy(acc[j], out_hbm.at[c, s * CPT + j])


def _dinv_row(degp):
    return lax.rsqrt(jnp.sum(degp, axis=0, keepdims=True) + 1.0)


def _tc_a_body(w1t_ref, x_ref, degp_ref, out_ref):
    dinv = _dinv_row(degp_ref[...])
    z = lax.dot_general(
        w1t_ref[...], x_ref[...], (((1,), (1,)), ((), ())),
        preferred_element_type=jnp.float32,
    )
    out_ref[...] = z * dinv


def _tc_b_body(hs1_ref, raw_ref, degp_ref, b1_ref, out_ref):
    i = pl.program_id(0)
    dinv = _dinv_row(degp_ref[...])
    h = jnp.maximum(dinv * (hs1_ref[...] + raw_ref[0] + raw_ref[1]) + b1_ref[...],
                    0.0)
    cols = i * BLK + lax.broadcasted_iota(jnp.int32, (1, BLK), 1)
    out_ref[...] = jnp.where(cols < N, dinv * h, 0.0)


def _tc_c_body(hs2_ref, raw_ref, degp_ref, w2t_ref, b2_ref, wfc_ref, bfc_ref,
               out_ref, accum):
    i = pl.program_id(0)
    dinv = _dinv_row(degp_ref[...])
    agg = dinv * (hs2_ref[...] + raw_ref[0] + raw_ref[1])
    z = lax.dot_general(
        w2t_ref[...], agg, (((1,), (0,)), ((), ())),
        preferred_element_type=jnp.float32,
    ) + b2_ref[...]
    r = jnp.maximum(z, 0.0)
    cols = i * BLK + lax.broadcasted_iota(jnp.int32, (1, BLK), 1)
    r = jnp.where(cols < N, r, 0.0)
    part = jnp.sum(r, axis=1, keepdims=True)

    @pl.when(i == 0)
    def _():
        accum[...] = jnp.zeros_like(accum)

    accum[...] += part

    @pl.when(i == pl.num_programs(0) - 1)
    def _():
        g = jnp.sum(accum[...] * wfc_ref[...], axis=0, keepdims=True)
        out_ref[...] = g / jnp.float32(N) + bfc_ref[...]


def kernel(x, edge_index, W1, b1, W2, b2, Wfc, bfc):
    x_pad = jnp.pad(x, ((0, NP - N), (0, 0)))
    pad = jnp.full((EPAD - E,), DUMMY, jnp.int32)
    src_flat = jnp.concatenate([edge_index[0], pad])
    dst_flat = jnp.concatenate([edge_index[1], pad])

    degp = _deg_kernel(dst_flat)

    grid = NP // BLK
    full = lambda shape: pl.BlockSpec(shape, lambda i: (0,) * len(shape))
    colT = pl.BlockSpec((D_HID, BLK), lambda i: (0, i))
    deg_blk = pl.BlockSpec((32, BLK), lambda i: (0, i))
    raw_blk = pl.BlockSpec((2, D_HID, BLK), lambda i: (0, 0, i))

    hs1T = pl.pallas_call(
        _tc_a_body,
        grid=(grid,),
        in_specs=[full((D_HID, D_IN)), pl.BlockSpec((BLK, D_IN), lambda i: (i, 0)),
                  deg_blk],
        out_specs=colT,
        out_shape=jax.ShapeDtypeStruct((D_HID, NP), jnp.float32),
    )(W1.T, x_pad, degp)

    raw1 = _agg_kernel(hs1T, src_flat, dst_flat)

    hs2T = pl.pallas_call(
        _tc_b_body,
        grid=(grid,),
        in_specs=[colT, raw_blk, deg_blk, full((D_HID, 1))],
        out_specs=colT,
        out_shape=jax.ShapeDtypeStruct((D_HID, NP), jnp.float32),
    )(hs1T, raw1, degp, b1.reshape(D_HID, 1))

    raw2 = _agg_kernel(hs2T, src_flat, dst_flat)

    out = pl.pallas_call(
        _tc_c_body,
        grid=(grid,),
        in_specs=[
            colT,
            raw_blk,
            deg_blk,
            full((D_OUT, D_HID)),
            full((D_OUT, 1)),
            full((D_OUT, 1)),
            full((1, 1)),
        ],
        out_specs=pl.BlockSpec((1, 1), lambda i: (0, 0)),
        out_shape=jax.ShapeDtypeStruct((1, 1), jnp.float32),
        scratch_shapes=[pltpu.VMEM((D_OUT, 1), jnp.float32)],
    )(hs2T, raw2, degp, W2.T, b2.reshape(D_OUT, 1), Wfc, bfc.reshape(1, 1))
    return out


# speedup vs baseline: 68.8552x; 1.0969x over previous
import functools

import jax
import jax.numpy as jnp
from jax import lax
from jax.experimental import pallas as pl
from jax.experimental.pallas import tpu as pltpu
from jax.experimental.pallas import tpu_sc as plsc

N = 10000
E = 320000
D_IN = 128
D_HID = 64
D_OUT = 128

BLK = 1000
CHK = 4000
NCH = 40
EPT = E // 32
CPT = 4

_mesh = plsc.VectorSubcoreMesh(core_axis_name="c", subcore_axis_name="s")
_params = pltpu.CompilerParams(needs_layout_passes=False)


@functools.partial(
    pl.kernel,
    mesh=_mesh,
    out_type=jax.ShapeDtypeStruct((32, N), jnp.float32),
    compiler_params=_params,
    scratch_types=[
        pltpu.VMEM((N,), jnp.float32),
        pltpu.VMEM((EPT,), jnp.int32),
    ],
)
def _deg_kernel(dst_hbm, out_hbm, acc, dv):
    c = lax.axis_index("c")
    s = lax.axis_index("s")
    t = c * 16 + s
    pltpu.sync_copy(dst_hbm.at[pl.ds(t * EPT, EPT)], dv)
    z = jnp.zeros((16,), jnp.float32)
    ones = jnp.ones((16,), jnp.float32)

    @functools.partial(plsc.parallel_loop, 0, N // 16, unroll=5)
    def _(i):
        acc[pl.ds(i * 16, 16)] = z

    @functools.partial(plsc.parallel_loop, 0, EPT // 16, unroll=5)
    def _(g):
        ov = dv[pl.ds(g * 16, 16)]
        plsc.addupdate_scatter(acc, [ov], ones)

    pltpu.sync_copy(acc, out_hbm.at[t])


@functools.partial(
    pl.kernel,
    mesh=_mesh,
    out_type=jax.ShapeDtypeStruct((2, D_HID, N), jnp.float32),
    compiler_params=_params,
    scratch_types=[
        [pltpu.VMEM((N,), jnp.float32)] * CPT,
        [pltpu.VMEM((N,), jnp.float32)] * CPT,
        [pltpu.VMEM((CHK,), jnp.int32)] * 2,
        [pltpu.VMEM((CHK,), jnp.int32)] * 2,
        [pltpu.SemaphoreType.DMA] * 2,
        [pltpu.SemaphoreType.DMA] * 2,
    ],
)
def _agg_kernel(hsT_hbm, src_hbm, dst_hbm, out_hbm, tbl, acc, sv, dv, ssem, dsem):
    c = lax.axis_index("c")
    s = lax.axis_index("s")
    z = jnp.zeros((16,), jnp.float32)
    half = c * (NCH * CHK)

    def stage(k, b):
        pltpu.async_copy(src_hbm.at[pl.ds(half + k * CHK, CHK)], sv[b], ssem[b])
        pltpu.async_copy(dst_hbm.at[pl.ds(half + k * CHK, CHK)], dv[b], dsem[b])

    def wait(b):
        pltpu.make_async_copy(src_hbm.at[pl.ds(0, CHK)], sv[b], ssem[b]).wait()
        pltpu.make_async_copy(dst_hbm.at[pl.ds(0, CHK)], dv[b], dsem[b]).wait()

    def process(b):
        @functools.partial(plsc.parallel_loop, 0, CHK // 16, unroll=10)
        def _(g):
            iv = sv[b][pl.ds(g * 16, 16)]
            ov = dv[b][pl.ds(g * 16, 16)]
            for j in range(CPT):
                vals = plsc.load_gather(tbl[j], [iv])
                plsc.addupdate_scatter(acc[j], [ov], vals)

    stage(0, 0)
    for j in range(CPT):
        pltpu.sync_copy(hsT_hbm.at[s * CPT + j], tbl[j])

    @functools.partial(plsc.parallel_loop, 0, N // 16, unroll=5)
    def _(i):
        for j in range(CPT):
            acc[j][pl.ds(i * 16, 16)] = z

    def pair(p, _):
        wait(0)
        stage(2 * p + 1, 1)
        process(0)
        wait(1)

        @pl.when(p + 1 < NCH // 2)
        def _():
            stage(2 * p + 2, 0)

        process(1)
        return 0

    lax.fori_loop(0, NCH // 2, pair, 0)
    for j in range(CPT):
        pltpu.sync_copy(acc[j], out_hbm.at[c, s * CPT + j])


def _dinv_row(degp):
    return lax.rsqrt(jnp.sum(degp, axis=0, keepdims=True) + 1.0)


def _tc_a_body(w1t_ref, x_ref, degp_ref, out_ref):
    dinv = _dinv_row(degp_ref[...])
    z = lax.dot_general(
        w1t_ref[...], x_ref[...], (((1,), (1,)), ((), ())),
        preferred_element_type=jnp.float32,
    )
    out_ref[...] = z * dinv


def _tc_b_body(hs1_ref, raw_ref, degp_ref, b1_ref, out_ref):
    dinv = _dinv_row(degp_ref[...])
    h = jnp.maximum(dinv * (hs1_ref[...] + raw_ref[0] + raw_ref[1]) + b1_ref[...],
                    0.0)
    out_ref[...] = dinv * h


def _tc_c_body(hs2_ref, raw_ref, degp_ref, w2t_ref, b2_ref, wfc_ref, bfc_ref,
               out_ref):
    dinv = _dinv_row(degp_ref[...])
    agg = dinv * (hs2_ref[...] + raw_ref[0] + raw_ref[1])
    z = lax.dot_general(
        w2t_ref[...], agg, (((1,), (0,)), ((), ())),
        preferred_element_type=jnp.float32,
    ) + b2_ref[...]
    r = jnp.maximum(z, 0.0)
    part = jnp.sum(r, axis=1, keepdims=True)
    g = jnp.sum(part * wfc_ref[...], axis=0, keepdims=True)
    out_ref[...] = g / jnp.float32(N) + bfc_ref[...]


def kernel(x, edge_index, W1, b1, W2, b2, Wfc, bfc):
    src_flat = edge_index[0]
    dst_flat = edge_index[1]

    degp = _deg_kernel(dst_flat)

    hs1T = pl.pallas_call(
        _tc_a_body,
        out_shape=jax.ShapeDtypeStruct((D_HID, N), jnp.float32),
    )(W1.T, x, degp)

    raw1 = _agg_kernel(hs1T, src_flat, dst_flat)

    hs2T = pl.pallas_call(
        _tc_b_body,
        out_shape=jax.ShapeDtypeStruct((D_HID, N), jnp.float32),
    )(hs1T, raw1, degp, b1.reshape(D_HID, 1))

    raw2 = _agg_kernel(hs2T, src_flat, dst_flat)

    out = pl.pallas_call(
        _tc_c_body,
        out_shape=jax.ShapeDtypeStruct((1, 1), jnp.float32),
    )(hs2T, raw2, degp, W2.T, b2.reshape(D_OUT, 1), Wfc, bfc.reshape(1, 1))
    return out
